# baseline (speedup 1.0000x reference)
import functools

import jax
import jax.numpy as jnp
from jax import lax
from jax.experimental import pallas as pl
from jax.experimental.pallas import tpu as pltpu
from jax.experimental.pallas import tpu_sc as plsc

B = 4096
H = 50
D = 64
N = B * H
NC = 2
NS = 16
NW = NC * NS
PER_W = N // NW
R = 128
CHUNKS = PER_W // R
NBUF = 2


def _sc_body(sid_hbm, fid_hbm, stab_hbm, ftab_hbm, out_hbm,
             sidx_v, fidx_v,
             s0, s1, f0, f1, out_v,
             sem_s0, sem_s1, sem_f0, sem_f1):
    wid = lax.axis_index("s") * NC + lax.axis_index("c")
    base = wid * PER_W

    sbufs = (s0, s1)
    fbufs = (f0, f1)
    ssems = (sem_s0, sem_s1)
    fsems = (sem_f0, sem_f1)

    pltpu.sync_copy(sid_hbm.at[pl.ds(wid * CHUNKS, CHUNKS)], sidx_v)
    pltpu.sync_copy(fid_hbm.at[pl.ds(wid * CHUNKS, CHUNKS)], fidx_v)

    def start(c, k):
        pltpu.async_copy(stab_hbm.at[sidx_v.at[c]], sbufs[k], ssems[k])
        pltpu.async_copy(ftab_hbm.at[fidx_v.at[c]], fbufs[k], fsems[k])

    def wait(c, k):
        pltpu.make_async_copy(stab_hbm.at[sidx_v.at[c]], sbufs[k],
                              ssems[k]).wait()
        pltpu.make_async_copy(ftab_hbm.at[fidx_v.at[c]], fbufs[k],
                              fsems[k]).wait()

    def compute(c, k):
        sbuf, fbuf = sbufs[k], fbufs[k]

        def group(g, carry):
            r0 = g * 16
            lane = lax.iota(jnp.int32, 16)
            acc = jnp.zeros((16,), jnp.float32)
            for j in range(16):
                r = r0 + j
                p16 = jnp.zeros((16,), jnp.float32)
                for q in range(2):
                    sa, sb = plsc.unpack(sbuf[r, pl.ds(q * 32, 32)],
                                         format=plsc.PackFormat.INTERLEAVED)
                    fa, fb = plsc.unpack(fbuf[r, pl.ds(q * 32, 32)],
                                         format=plsc.PackFormat.INTERLEAVED)
                    p16 = p16 + sa * fa + sb * fb
                acc = jnp.where(lane == j, jnp.sum(p16), acc)
            out_v[pl.ds(c * R + r0, 16)] = acc
            return carry

        lax.fori_loop(0, R // 16, group, 0)

    for k in range(NBUF):
        start(k, k)

    def quad(i, carry):
        for k in range(NBUF):
            c = NBUF * i + k
            wait(c, k)
            compute(c, k)
            start(c + NBUF, k)
        return carry

    lax.fori_loop(0, (CHUNKS - NBUF) // NBUF, quad, 0)

    for k in range(NBUF):
        c = CHUNKS - NBUF + k
        wait(c, k)
        compute(c, k)

    pltpu.sync_copy(out_v, out_hbm.at[pl.ds(base, PER_W)])


@jax.jit
def kernel(sample_id, filename, sample_table, filename_table):
    sid = sample_id.reshape(NW * CHUNKS, R).astype(jnp.int32)
    fid = filename.reshape(NW * CHUNKS, R).astype(jnp.int32)
    mesh = plsc.VectorSubcoreMesh(core_axis_name="c", subcore_axis_name="s")
    run = pl.kernel(
        _sc_body,
        out_type=jax.ShapeDtypeStruct((N,), jnp.float32),
        mesh=mesh,
        scratch_types=[
            pltpu.VMEM((CHUNKS, R), jnp.int32),
            pltpu.VMEM((CHUNKS, R), jnp.int32),
        ] + [pltpu.VMEM((R, D), jnp.bfloat16)] * (2 * NBUF) + [
            pltpu.VMEM((PER_W,), jnp.float32),
        ] + [pltpu.SemaphoreType.DMA] * (2 * NBUF),
        compiler_params=pltpu.CompilerParams(
            needs_layout_passes=False, use_tc_tiling_on_sc=False),
    )
    out = run(sid, fid, sample_table.astype(jnp.bfloat16),
              filename_table.astype(jnp.bfloat16))
    return out.reshape(B, H)


# speedup vs baseline: 1.1905x; 1.1905x over previous
import functools

import jax
import jax.numpy as jnp
from jax import lax
from jax.experimental import pallas as pl
from jax.experimental.pallas import tpu as pltpu
from jax.experimental.pallas import tpu_sc as plsc

B = 4096
H = 50
D = 64
N = B * H
NC = 2
NS = 16
NW = NC * NS
PER_W = N // NW
R = 320
CHUNKS = PER_W // R


def _sc_body(sid_hbm, fid_hbm, stab_hbm, ftab_hbm, out_hbm,
             sidx_v, fidx_v, s0, s1, f0, f1, out_v,
             sem_s0, sem_s1, sem_f0, sem_f1):
    wid = lax.axis_index("s") * NC + lax.axis_index("c")
    base = wid * PER_W

    pltpu.sync_copy(sid_hbm.at[pl.ds(wid * CHUNKS, CHUNKS)], sidx_v)
    pltpu.sync_copy(fid_hbm.at[pl.ds(wid * CHUNKS, CHUNKS)], fidx_v)

    def start(c, sbuf, fbuf, ssem, fsem):
        pltpu.async_copy(stab_hbm.at[sidx_v.at[c]], sbuf, ssem)
        pltpu.async_copy(ftab_hbm.at[fidx_v.at[c]], fbuf, fsem)

    def wait(c, sbuf, fbuf, ssem, fsem):
        pltpu.make_async_copy(stab_hbm.at[sidx_v.at[c]], sbuf, ssem).wait()
        pltpu.make_async_copy(ftab_hbm.at[fidx_v.at[c]], fbuf, fsem).wait()

    def compute(c, sbuf, fbuf):
        def group(g, carry):
            r0 = g * 16
            lane = lax.iota(jnp.int32, 16)
            acc = jnp.zeros((16,), jnp.float32)
            for j in range(16):
                r = r0 + j
                p = (sbuf[r, pl.ds(0, 16)] * fbuf[r, pl.ds(0, 16)]
                     + sbuf[r, pl.ds(16, 16)] * fbuf[r, pl.ds(16, 16)]
                     + sbuf[r, pl.ds(32, 16)] * fbuf[r, pl.ds(32, 16)]
                     + sbuf[r, pl.ds(48, 16)] * fbuf[r, pl.ds(48, 16)])
                acc = jnp.where(lane == j, jnp.sum(p), acc)
            out_v[pl.ds(c * R + r0, 16)] = acc
            return carry

        lax.fori_loop(0, R // 16, group, 0)

    start(0, s0, f0, sem_s0, sem_f0)
    start(1, s1, f1, sem_s1, sem_f1)

    def pair(i, carry):
        c0 = 2 * i
        wait(c0, s0, f0, sem_s0, sem_f0)
        compute(c0, s0, f0)
        start(c0 + 2, s0, f0, sem_s0, sem_f0)
        c1 = 2 * i + 1
        wait(c1, s1, f1, sem_s1, sem_f1)
        compute(c1, s1, f1)
        start(c1 + 2, s1, f1, sem_s1, sem_f1)
        return carry

    lax.fori_loop(0, (CHUNKS - 2) // 2, pair, 0)

    wait(CHUNKS - 2, s0, f0, sem_s0, sem_f0)
    compute(CHUNKS - 2, s0, f0)
    wait(CHUNKS - 1, s1, f1, sem_s1, sem_f1)
    compute(CHUNKS - 1, s1, f1)

    pltpu.sync_copy(out_v, out_hbm.at[pl.ds(base, PER_W)])


@jax.jit
def kernel(sample_id, filename, sample_table, filename_table):
    sid = sample_id.reshape(NW * CHUNKS, R).astype(jnp.int32)
    fid = filename.reshape(NW * CHUNKS, R).astype(jnp.int32)
    mesh = plsc.VectorSubcoreMesh(core_axis_name="c", subcore_axis_name="s")
    run = pl.kernel(
        _sc_body,
        out_type=jax.ShapeDtypeStruct((N,), jnp.float32),
        mesh=mesh,
        scratch_types=[
            pltpu.VMEM((CHUNKS, R), jnp.int32),
            pltpu.VMEM((CHUNKS, R), jnp.int32),
            pltpu.VMEM((R, D), jnp.float32),
            pltpu.VMEM((R, D), jnp.float32),
            pltpu.VMEM((R, D), jnp.float32),
            pltpu.VMEM((R, D), jnp.float32),
            pltpu.VMEM((PER_W,), jnp.float32),
            pltpu.SemaphoreType.DMA,
            pltpu.SemaphoreType.DMA,
            pltpu.SemaphoreType.DMA,
            pltpu.SemaphoreType.DMA,
        ],
        compiler_params=pltpu.CompilerParams(
            needs_layout_passes=False, use_tc_tiling_on_sc=False),
    )
    out = run(sid, fid, sample_table, filename_table)
    return out.reshape(B, H)


# speedup vs baseline: 1.1919x; 1.0011x over previous
import functools

import jax
import jax.numpy as jnp
from jax import lax
from jax.experimental import pallas as pl
from jax.experimental.pallas import tpu as pltpu
from jax.experimental.pallas import tpu_sc as plsc

B = 4096
H = 50
D = 64
N = B * H
NC = 2
NS = 16
NW = NC * NS
PER_W = N // NW
R = 400
CHUNKS = PER_W // R


def _sc_body(sid_hbm, fid_hbm, stab_hbm, ftab_hbm, out_hbm,
             sidx_v, fidx_v, s0, s1, f0, f1, out_v,
             sem_s0, sem_s1, sem_f0, sem_f1):
    wid = lax.axis_index("s") * NC + lax.axis_index("c")
    base = wid * PER_W

    pltpu.sync_copy(sid_hbm.at[pl.ds(wid * CHUNKS, CHUNKS)], sidx_v)
    pltpu.sync_copy(fid_hbm.at[pl.ds(wid * CHUNKS, CHUNKS)], fidx_v)

    def start(c, sbuf, fbuf, ssem, fsem):
        pltpu.async_copy(stab_hbm.at[sidx_v.at[c]], sbuf, ssem)
        pltpu.async_copy(ftab_hbm.at[fidx_v.at[c]], fbuf, fsem)

    def wait(c, sbuf, fbuf, ssem, fsem):
        pltpu.make_async_copy(stab_hbm.at[sidx_v.at[c]], sbuf, ssem).wait()
        pltpu.make_async_copy(ftab_hbm.at[fidx_v.at[c]], fbuf, fsem).wait()

    def compute(c, sbuf, fbuf):
        def group(g, carry):
            r0 = g * 16
            lane = lax.iota(jnp.int32, 16)
            acc = jnp.zeros((16,), jnp.float32)
            for j in range(16):
                r = r0 + j
                p = (sbuf[r, pl.ds(0, 16)] * fbuf[r, pl.ds(0, 16)]
                     + sbuf[r, pl.ds(16, 16)] * fbuf[r, pl.ds(16, 16)]
                     + sbuf[r, pl.ds(32, 16)] * fbuf[r, pl.ds(32, 16)]
                     + sbuf[r, pl.ds(48, 16)] * fbuf[r, pl.ds(48, 16)])
                acc = jnp.where(lane == j, jnp.sum(p), acc)
            out_v[pl.ds(c * R + r0, 16)] = acc
            return carry

        lax.fori_loop(0, R // 16, group, 0)

    start(0, s0, f0, sem_s0, sem_f0)
    start(1, s1, f1, sem_s1, sem_f1)

    def pair(i, carry):
        c0 = 2 * i
        wait(c0, s0, f0, sem_s0, sem_f0)
        compute(c0, s0, f0)
        start(c0 + 2, s0, f0, sem_s0, sem_f0)
        c1 = 2 * i + 1
        wait(c1, s1, f1, sem_s1, sem_f1)
        compute(c1, s1, f1)
        start(c1 + 2, s1, f1, sem_s1, sem_f1)
        return carry

    lax.fori_loop(0, (CHUNKS - 2) // 2, pair, 0)

    wait(CHUNKS - 2, s0, f0, sem_s0, sem_f0)
    compute(CHUNKS - 2, s0, f0)
    wait(CHUNKS - 1, s1, f1, sem_s1, sem_f1)
    compute(CHUNKS - 1, s1, f1)

    pltpu.sync_copy(out_v, out_hbm.at[pl.ds(base, PER_W)])


@jax.jit
def kernel(sample_id, filename, sample_table, filename_table):
    sid = sample_id.reshape(NW * CHUNKS, R).astype(jnp.int32)
    fid = filename.reshape(NW * CHUNKS, R).astype(jnp.int32)
    mesh = plsc.VectorSubcoreMesh(core_axis_name="c", subcore_axis_name="s")
    run = pl.kernel(
        _sc_body,
        out_type=jax.ShapeDtypeStruct((N,), jnp.float32),
        mesh=mesh,
        scratch_types=[
            pltpu.VMEM((CHUNKS, R), jnp.int32),
            pltpu.VMEM((CHUNKS, R), jnp.int32),
            pltpu.VMEM((R, D), jnp.float32),
            pltpu.VMEM((R, D), jnp.float32),
            pltpu.VMEM((R, D), jnp.float32),
            pltpu.VMEM((R, D), jnp.float32),
            pltpu.VMEM((PER_W,), jnp.float32),
            pltpu.SemaphoreType.DMA,
            pltpu.SemaphoreType.DMA,
            pltpu.SemaphoreType.DMA,
            pltpu.SemaphoreType.DMA,
        ],
        compiler_params=pltpu.CompilerParams(
            needs_layout_passes=False, use_tc_tiling_on_sc=False),
    )
    out = run(sid, fid, sample_table, filename_table)
    return out.reshape(B, H)
